# baseline (speedup 1.0000x reference)
import jax
import jax.numpy as jnp
from jax.experimental import pallas as pl
from jax.experimental.pallas import tpu as pltpu

_BM = 400


def _xw1_kernel(x_ref, w1_ref, g_ref):
    g_ref[...] = jnp.dot(
        x_ref[...].astype(jnp.bfloat16),
        w1_ref[...].astype(jnp.bfloat16),
        preferred_element_type=jnp.float32,
    ).astype(jnp.bfloat16)


def _agg1_kernel(adj_ref, g_ref, b1_ref, h_ref):
    acc = jnp.dot(
        adj_ref[...].astype(jnp.bfloat16),
        g_ref[...],
        preferred_element_type=jnp.float32,
    )
    h_ref[...] = jnp.maximum(acc + b1_ref[...], 0.0)


def _hw2_kernel(h_ref, w2_ref, p_ref):
    p_ref[...] = jnp.dot(
        h_ref[...].astype(jnp.bfloat16),
        w2_ref[...].astype(jnp.bfloat16),
        preferred_element_type=jnp.float32,
    ).astype(jnp.bfloat16)


def _agg2_kernel(adj_ref, p_ref, b2_ref, out_ref):
    acc = jnp.dot(
        adj_ref[...].astype(jnp.bfloat16),
        p_ref[...],
        preferred_element_type=jnp.float32,
    )
    out_ref[...] = acc + b2_ref[...]


def kernel(x, adj, W1, b1, W2, b2):
    n, nfeat = x.shape
    nhid = W1.shape[1]
    nclass = W2.shape[1]
    bm = _BM if n % _BM == 0 else n
    nb = n // bm

    b1r = b1.reshape(1, nhid)
    b2r = b2.reshape(1, nclass)

    g = pl.pallas_call(
        _xw1_kernel,
        out_shape=jax.ShapeDtypeStruct((n, nhid), jnp.bfloat16),
    )(x, W1)

    h = pl.pallas_call(
        _agg1_kernel,
        grid=(nb,),
        in_specs=[
            pl.BlockSpec((bm, n), lambda i: (i, 0)),
            pl.BlockSpec((n, nhid), lambda i: (0, 0)),
            pl.BlockSpec((1, nhid), lambda i: (0, 0)),
        ],
        out_specs=pl.BlockSpec((bm, nhid), lambda i: (i, 0)),
        out_shape=jax.ShapeDtypeStruct((n, nhid), jnp.float32),
    )(adj, g, b1r)

    p = pl.pallas_call(
        _hw2_kernel,
        out_shape=jax.ShapeDtypeStruct((n, nclass), jnp.bfloat16),
    )(h, W2)

    out = pl.pallas_call(
        _agg2_kernel,
        grid=(nb,),
        in_specs=[
            pl.BlockSpec((bm, n), lambda i: (i, 0)),
            pl.BlockSpec((n, nclass), lambda i: (0, 0)),
            pl.BlockSpec((1, nclass), lambda i: (0, 0)),
        ],
        out_specs=pl.BlockSpec((bm, nclass), lambda i: (i, 0)),
        out_shape=jax.ShapeDtypeStruct((n, nclass), jnp.float32),
    )(adj, p, b2r)

    return out
